# Initial kernel scaffold; baseline (speedup 1.0000x reference)
import jax
import jax.numpy as jnp
from jax import lax
from jax.experimental import pallas as pl
from jax.experimental.pallas import tpu as pltpu
from jax.experimental.pallas import tpu_sc as plsc

N = 10000
E = 320000
D = 128
NC = 2
NS = 16
NW = NC * NS
EPW = E // NW
CH = 80
NCH = EPW // CH
RPT = N // NS


def _sc_agg_body(y_hbm, src_hbm, dst_hbm, sum_hbm, cnt_hbm,
                 src_v, dst_v, stage, ones_v, zrow, zcnt, acc_sh, cnt_sh,
                 gsem):
    c = lax.axis_index("c")
    s = lax.axis_index("s")
    wid = c * NS + s

    def zr(r, carry):
        def zk(k, carry2):
            zrow[r, pl.ds(k * 16, 16)] = jnp.zeros((16,), jnp.float32)
            return carry2
        return lax.fori_loop(0, D // 16, zk, carry)
    lax.fori_loop(0, 125, zr, 0)

    def zc(r, carry):
        zcnt[r, pl.ds(0, 16)] = jnp.zeros((16,), jnp.float32)
        return carry
    lax.fori_loop(0, RPT, zc, 0)

    def oc(r, carry):
        ones_v[r, pl.ds(0, 16)] = jnp.ones((16,), jnp.float32)
        return carry
    lax.fori_loop(0, CH, oc, 0)

    for t in range(5):
        pltpu.sync_copy(zrow, acc_sh.at[pl.ds(s * RPT + t * 125, 125)])
    pltpu.sync_copy(zcnt, cnt_sh.at[pl.ds(s * RPT, RPT)])
    plsc.subcore_barrier()

    pltpu.sync_copy(src_hbm.at[wid], src_v)
    pltpu.sync_copy(dst_hbm.at[wid], dst_v)

    def chunk(j, carry):
        pltpu.async_copy(y_hbm.at[src_v.at[j]], stage, gsem).wait()
        pltpu.sync_copy(stage, acc_sh.at[dst_v.at[j]], add=True)
        pltpu.sync_copy(ones_v, cnt_sh.at[dst_v.at[j]], add=True)
        return carry
    lax.fori_loop(0, NCH, chunk, 0)

    plsc.subcore_barrier()

    for t in range(5):
        pltpu.sync_copy(acc_sh.at[pl.ds(s * RPT + t * 125, 125)],
                        sum_hbm.at[c, pl.ds(s * RPT + t * 125, 125)])
    pltpu.sync_copy(cnt_sh.at[pl.ds(s * RPT, RPT)],
                    cnt_hbm.at[c, pl.ds(s * RPT, RPT)])


def _sc_agg(y, src_r, dst_r):
    mesh = plsc.VectorSubcoreMesh(core_axis_name="c", subcore_axis_name="s")
    return pl.kernel(
        _sc_agg_body,
        out_type=(jax.ShapeDtypeStruct((NC, N, D), jnp.float32),
                  jax.ShapeDtypeStruct((NC, N, 16), jnp.float32)),
        mesh=mesh,
        scratch_types=[
            pltpu.VMEM((NCH, CH), jnp.int32),
            pltpu.VMEM((NCH, CH), jnp.int32),
            pltpu.VMEM((CH, D), jnp.float32),
            pltpu.VMEM((CH, 16), jnp.float32),
            pltpu.VMEM((125, D), jnp.float32),
            pltpu.VMEM((RPT, 16), jnp.float32),
            pltpu.VMEM_SHARED((N, D), jnp.float32),
            pltpu.VMEM_SHARED((N, 16), jnp.float32),
            pltpu.SemaphoreType.DMA,
        ],
    )(y, src_r, dst_r)


_BLK = 1000
_GRID = N // _BLK


def _tc1_body(x_ref, wl_ref, wr_ref, b_ref, y_ref, r_ref):
    xb = x_ref[...]
    y_ref[...] = jnp.dot(xb, wl_ref[...], preferred_element_type=jnp.float32,
                         precision=lax.Precision.HIGHEST)
    r_ref[...] = jnp.dot(xb, wr_ref[...], preferred_element_type=jnp.float32,
                         precision=lax.Precision.HIGHEST) + b_ref[...]


def _tc1(x, wl_t, wr_t, b):
    return pl.pallas_call(
        _tc1_body,
        grid=(_GRID,),
        in_specs=[pl.BlockSpec((_BLK, D), lambda i: (i, 0)),
                  pl.BlockSpec((D, D), lambda i: (0, 0)),
                  pl.BlockSpec((D, D), lambda i: (0, 0)),
                  pl.BlockSpec((1, D), lambda i: (0, 0))],
        out_specs=[pl.BlockSpec((_BLK, D), lambda i: (i, 0)),
                   pl.BlockSpec((_BLK, D), lambda i: (i, 0))],
        out_shape=[jax.ShapeDtypeStruct((N, D), jnp.float32),
                   jax.ShapeDtypeStruct((N, D), jnp.float32)],
    )(x, wl_t, wr_t, b)


def _tc2_body(sa_ref, sb_ref, ca_ref, cb_ref, r1_ref, wl_ref, wr_ref, b_ref,
              y_ref, r_ref):
    cnt = ca_ref[...][:, 0:1] + cb_ref[...][:, 0:1]
    mean = (sa_ref[...] + sb_ref[...]) / jnp.maximum(cnt, 1.0)
    h = jnp.maximum(mean + r1_ref[...], 0.0)
    y_ref[...] = jnp.dot(h, wl_ref[...], preferred_element_type=jnp.float32,
                         precision=lax.Precision.HIGHEST)
    r_ref[...] = jnp.dot(h, wr_ref[...], preferred_element_type=jnp.float32,
                         precision=lax.Precision.HIGHEST) + b_ref[...]


def _tc2(sa, sb, ca, cb, r1, wl_t, wr_t, b):
    return pl.pallas_call(
        _tc2_body,
        grid=(_GRID,),
        in_specs=[pl.BlockSpec((_BLK, D), lambda i: (i, 0)),
                  pl.BlockSpec((_BLK, D), lambda i: (i, 0)),
                  pl.BlockSpec((_BLK, 16), lambda i: (i, 0)),
                  pl.BlockSpec((_BLK, 16), lambda i: (i, 0)),
                  pl.BlockSpec((_BLK, D), lambda i: (i, 0)),
                  pl.BlockSpec((D, D), lambda i: (0, 0)),
                  pl.BlockSpec((D, D), lambda i: (0, 0)),
                  pl.BlockSpec((1, D), lambda i: (0, 0))],
        out_specs=[pl.BlockSpec((_BLK, D), lambda i: (i, 0)),
                   pl.BlockSpec((_BLK, D), lambda i: (i, 0))],
        out_shape=[jax.ShapeDtypeStruct((N, D), jnp.float32),
                   jax.ShapeDtypeStruct((N, D), jnp.float32)],
    )(sa, sb, ca, cb, r1, wl_t, wr_t, b)


def _tc3_body(sa_ref, sb_ref, ca_ref, cb_ref, r2_ref, o_ref):
    cnt = ca_ref[...][:, 0:1] + cb_ref[...][:, 0:1]
    mean = (sa_ref[...] + sb_ref[...]) / jnp.maximum(cnt, 1.0)
    z = mean + r2_ref[...]
    o_ref[...] = 1.0 / (1.0 + jnp.exp(-z))


def _tc3(sa, sb, ca, cb, r2):
    return pl.pallas_call(
        _tc3_body,
        grid=(_GRID,),
        in_specs=[pl.BlockSpec((_BLK, D), lambda i: (i, 0)),
                  pl.BlockSpec((_BLK, D), lambda i: (i, 0)),
                  pl.BlockSpec((_BLK, 16), lambda i: (i, 0)),
                  pl.BlockSpec((_BLK, 16), lambda i: (i, 0)),
                  pl.BlockSpec((_BLK, D), lambda i: (i, 0))],
        out_specs=pl.BlockSpec((_BLK, D), lambda i: (i, 0)),
        out_shape=jax.ShapeDtypeStruct((N, D), jnp.float32),
    )(sa, sb, ca, cb, r2)


def kernel(x, edge_index, W1_l, b1, W1_r, W2_l, b2, W2_r):
    src_r = edge_index[0].reshape(NW, NCH, CH)
    dst_r = edge_index[1].reshape(NW, NCH, CH)
    y1, r1 = _tc1(x, W1_l.T, W1_r.T, b1.reshape(1, D))
    s1, c1 = _sc_agg(y1, src_r, dst_r)
    y2, r2 = _tc2(s1[0], s1[1], c1[0], c1[1], r1, W2_l.T, W2_r.T,
                  b2.reshape(1, D))
    s2, _c2 = _sc_agg(y2, src_r, dst_r)
    return _tc3(s2[0], s2[1], c1[0], c1[1], r2)


# speedup vs baseline: 6.9082x; 6.9082x over previous
import jax
import jax.numpy as jnp
from jax import lax
from jax.experimental import pallas as pl
from jax.experimental.pallas import tpu as pltpu
from jax.experimental.pallas import tpu_sc as plsc

N = 10000
E = 320000
D = 128
NC = 2
NS = 16
NW = NC * NS
EPW = E // NW
CH = 80
NCH = EPW // CH
RPT = 632
NP = NS * RPT


def _sc_agg_body(y_hbm, src_hbm, dst_hbm, zacc_hbm,
                 sum_hbm, cnt_hbm,
                 src_v, dst_v, stage, ones_v, ctmp_v, acc_sh, cnt_sh, gsem):
    c = lax.axis_index("c")
    s = lax.axis_index("s")
    wid = c * NS + s
    base = pl.multiple_of(s * RPT, 8)

    for k in range(CH // 16):
        ones_v[pl.ds(k * 16, 16)] = jnp.ones((16,), jnp.float32)
    for k in range(RPT // 16 + 1):
        ctmp_v[pl.ds(k * 16, 16)] = jnp.zeros((16,), jnp.float32)

    pltpu.sync_copy(zacc_hbm.at[pl.ds(base, RPT)], acc_sh.at[pl.ds(base, RPT)])
    pltpu.sync_copy(ctmp_v.at[pl.ds(0, RPT)], cnt_sh.at[pl.ds(base, RPT)])

    pltpu.sync_copy(src_hbm.at[wid], src_v)
    pltpu.sync_copy(dst_hbm.at[wid], dst_v)
    plsc.subcore_barrier()

    def chunk(j, carry):
        pltpu.async_copy(y_hbm.at[src_v.at[j]], stage, gsem).wait()
        pltpu.sync_copy(stage, acc_sh.at[dst_v.at[j]], add=True)
        pltpu.sync_copy(ones_v, cnt_sh.at[dst_v.at[j]], add=True)
        return carry
    lax.fori_loop(0, NCH, chunk, 0)

    plsc.subcore_barrier()

    pltpu.sync_copy(acc_sh.at[pl.ds(base, RPT)],
                    sum_hbm.at[c, pl.ds(base, RPT)])
    pltpu.sync_copy(cnt_sh.at[pl.ds(base, RPT)], ctmp_v.at[pl.ds(0, RPT)])
    pltpu.sync_copy(ctmp_v.at[pl.ds(0, RPT)],
                    cnt_hbm.at[pl.ds(c * NP + base, RPT)])


def _sc_agg(y, src_r, dst_r, zacc):
    mesh = plsc.VectorSubcoreMesh(core_axis_name="c", subcore_axis_name="s")
    return pl.kernel(
        _sc_agg_body,
        out_type=(jax.ShapeDtypeStruct((NC, NP, D), jnp.float32),
                  jax.ShapeDtypeStruct((NC * NP,), jnp.float32)),
        mesh=mesh,
        scratch_types=[
            pltpu.VMEM((NCH, CH), jnp.int32),
            pltpu.VMEM((NCH, CH), jnp.int32),
            pltpu.VMEM((CH, D), jnp.float32),
            pltpu.VMEM((CH,), jnp.float32),
            pltpu.VMEM((RPT + 16,), jnp.float32),
            pltpu.VMEM_SHARED((NP, D), jnp.float32),
            pltpu.VMEM_SHARED((NP,), jnp.float32),
            pltpu.SemaphoreType.DMA,
        ],
    )(y, src_r, dst_r, zacc)


_BLK = 1000
_GRID = N // _BLK


def _tc1_body(x_ref, wl_ref, wr_ref, b_ref, y_ref, r_ref):
    xb = x_ref[...]
    y_ref[...] = jnp.dot(xb, wl_ref[...], preferred_element_type=jnp.float32,
                         precision=lax.Precision.HIGHEST)
    r_ref[...] = jnp.dot(xb, wr_ref[...], preferred_element_type=jnp.float32,
                         precision=lax.Precision.HIGHEST) + b_ref[...]


def _tc1(x, wl_t, wr_t, b):
    return pl.pallas_call(
        _tc1_body,
        grid=(_GRID,),
        in_specs=[pl.BlockSpec((_BLK, D), lambda i: (i, 0)),
                  pl.BlockSpec((D, D), lambda i: (0, 0)),
                  pl.BlockSpec((D, D), lambda i: (0, 0)),
                  pl.BlockSpec((1, D), lambda i: (0, 0))],
        out_specs=[pl.BlockSpec((_BLK, D), lambda i: (i, 0)),
                   pl.BlockSpec((_BLK, D), lambda i: (i, 0))],
        out_shape=[jax.ShapeDtypeStruct((N, D), jnp.float32),
                   jax.ShapeDtypeStruct((N, D), jnp.float32)],
    )(x, wl_t, wr_t, b)


def _tc2_body(sa_ref, sb_ref, ca_ref, cb_ref, r1_ref, wl_ref, wr_ref, b_ref,
              y_ref, r_ref):
    cnt = ca_ref[...] + cb_ref[...]
    mean = (sa_ref[...] + sb_ref[...]) / jnp.maximum(cnt, 1.0)
    h = jnp.maximum(mean + r1_ref[...], 0.0)
    y_ref[...] = jnp.dot(h, wl_ref[...], preferred_element_type=jnp.float32,
                         precision=lax.Precision.HIGHEST)
    r_ref[...] = jnp.dot(h, wr_ref[...], preferred_element_type=jnp.float32,
                         precision=lax.Precision.HIGHEST) + b_ref[...]


def _tc2(sa, sb, ca, cb, r1, wl_t, wr_t, b):
    return pl.pallas_call(
        _tc2_body,
        grid=(_GRID,),
        in_specs=[pl.BlockSpec((_BLK, D), lambda i: (i, 0)),
                  pl.BlockSpec((_BLK, D), lambda i: (i, 0)),
                  pl.BlockSpec((_BLK, 1), lambda i: (i, 0)),
                  pl.BlockSpec((_BLK, 1), lambda i: (i, 0)),
                  pl.BlockSpec((_BLK, D), lambda i: (i, 0)),
                  pl.BlockSpec((D, D), lambda i: (0, 0)),
                  pl.BlockSpec((D, D), lambda i: (0, 0)),
                  pl.BlockSpec((1, D), lambda i: (0, 0))],
        out_specs=[pl.BlockSpec((_BLK, D), lambda i: (i, 0)),
                   pl.BlockSpec((_BLK, D), lambda i: (i, 0))],
        out_shape=[jax.ShapeDtypeStruct((N, D), jnp.float32),
                   jax.ShapeDtypeStruct((N, D), jnp.float32)],
    )(sa, sb, ca, cb, r1, wl_t, wr_t, b)


def _tc3_body(sa_ref, sb_ref, ca_ref, cb_ref, r2_ref, o_ref):
    cnt = ca_ref[...] + cb_ref[...]
    mean = (sa_ref[...] + sb_ref[...]) / jnp.maximum(cnt, 1.0)
    z = mean + r2_ref[...]
    o_ref[...] = 1.0 / (1.0 + jnp.exp(-z))


def _tc3(sa, sb, ca, cb, r2):
    return pl.pallas_call(
        _tc3_body,
        grid=(_GRID,),
        in_specs=[pl.BlockSpec((_BLK, D), lambda i: (i, 0)),
                  pl.BlockSpec((_BLK, D), lambda i: (i, 0)),
                  pl.BlockSpec((_BLK, 1), lambda i: (i, 0)),
                  pl.BlockSpec((_BLK, 1), lambda i: (i, 0)),
                  pl.BlockSpec((_BLK, D), lambda i: (i, 0))],
        out_specs=pl.BlockSpec((_BLK, D), lambda i: (i, 0)),
        out_shape=jax.ShapeDtypeStruct((N, D), jnp.float32),
    )(sa, sb, ca, cb, r2)


def kernel(x, edge_index, W1_l, b1, W1_r, W2_l, b2, W2_r):
    src_r = edge_index[0].reshape(NW, NCH, CH)
    dst_r = edge_index[1].reshape(NW, NCH, CH)
    zacc = jnp.zeros((NP, D), jnp.float32)
    y1, r1 = _tc1(x, W1_l.T, W1_r.T, b1.reshape(1, D))
    s1, cnt = _sc_agg(y1, src_r, dst_r, zacc)
    ca = cnt[:NP].reshape(NP, 1)
    cb = cnt[NP:].reshape(NP, 1)
    y2, r2 = _tc2(s1[0], s1[1], ca, cb, r1, W2_l.T, W2_r.T,
                  b2.reshape(1, D))
    s2, _cnt2 = _sc_agg(y2, src_r, dst_r, zacc)
    return _tc3(s2[0], s2[1], ca, cb, r2)


# speedup vs baseline: 7.9743x; 1.1543x over previous
import functools

import jax
import jax.numpy as jnp
from jax import lax
from jax.experimental import pallas as pl
from jax.experimental.pallas import tpu as pltpu
from jax.experimental.pallas import tpu_sc as plsc

N = 10000
E = 320000
D = 128
NC = 2
NS = 16
NW = NC * NS
EPW = 10240
CH = 128
NCH = EPW // CH
EP = NW * EPW
RPT = 632
NP = NS * RPT


def _sc_agg_body(with_cnt, y_hbm, edges_hbm, zacc_hbm,
                 sum_hbm, cnt_hbm,
                 src_v, dst_v, stage, ones_v, ctmp_v, acc_sh, cnt_sh,
                 gsem, ssem):
    c = lax.axis_index("c")
    s = lax.axis_index("s")
    wid = c * NS + s
    base = pl.multiple_of(s * RPT, 8)

    for k in range(CH // 16):
        ones_v[pl.ds(k * 16, 16)] = jnp.ones((16,), jnp.float32)
    for k in range(RPT // 16 + 1):
        ctmp_v[pl.ds(k * 16, 16)] = jnp.zeros((16,), jnp.float32)

    pltpu.sync_copy(zacc_hbm.at[pl.ds(base, RPT)], acc_sh.at[pl.ds(base, RPT)])
    if with_cnt:
        pltpu.sync_copy(ctmp_v.at[pl.ds(0, RPT)], cnt_sh.at[pl.ds(base, RPT)])

    pltpu.sync_copy(edges_hbm.at[0, wid], src_v)
    pltpu.sync_copy(edges_hbm.at[1, wid], dst_v)
    plsc.subcore_barrier()

    def chunk(j, carry):
        pltpu.async_copy(y_hbm.at[src_v.at[j]], stage, gsem).wait()
        pltpu.sync_copy(stage, acc_sh.at[dst_v.at[j]], add=True)
        if with_cnt:
            pltpu.sync_copy(ones_v, cnt_sh.at[dst_v.at[j]], add=True)
        return carry
    lax.fori_loop(0, NCH, chunk, 0)

    plsc.subcore_barrier()

    pltpu.sync_copy(acc_sh.at[pl.ds(base, RPT)],
                    sum_hbm.at[c, pl.ds(base, RPT)])
    if with_cnt:
        pltpu.sync_copy(cnt_sh.at[pl.ds(base, RPT)], ctmp_v.at[pl.ds(0, RPT)])
        pltpu.sync_copy(ctmp_v.at[pl.ds(0, RPT)],
                        cnt_hbm.at[pl.ds(c * NP + base, RPT)])


def _sc_agg(y, edges_r, zacc, with_cnt):
    mesh = plsc.VectorSubcoreMesh(core_axis_name="c", subcore_axis_name="s")
    cnt_elems = NC * NP if with_cnt else 8
    out = pl.kernel(
        functools.partial(_sc_agg_body, with_cnt),
        out_type=(jax.ShapeDtypeStruct((NC, NP, D), jnp.float32),
                  jax.ShapeDtypeStruct((cnt_elems,), jnp.float32)),
        mesh=mesh,
        scratch_types=[
            pltpu.VMEM((NCH, CH), jnp.int32),
            pltpu.VMEM((NCH, CH), jnp.int32),
            pltpu.VMEM((CH, D), jnp.float32),
            pltpu.VMEM((CH,), jnp.float32),
            pltpu.VMEM((RPT + 16,), jnp.float32),
            pltpu.VMEM_SHARED((NP, D), jnp.float32),
            pltpu.VMEM_SHARED((NP,), jnp.float32),
            pltpu.SemaphoreType.DMA,
            pltpu.SemaphoreType.DMA,
        ],
    )(y, edges_r, zacc)
    return out


_BLK = 1000
_GRID = N // _BLK


def _tc1_body(x_ref, wl_ref, wr_ref, b_ref, y_ref, r_ref):
    xb = x_ref[...]
    y_ref[...] = jnp.dot(xb, wl_ref[...], preferred_element_type=jnp.float32,
                         precision=lax.Precision.HIGHEST)
    r_ref[...] = jnp.dot(xb, wr_ref[...], preferred_element_type=jnp.float32,
                         precision=lax.Precision.HIGHEST) + b_ref[...]


def _tc1(x, wl_t, wr_t, b):
    return pl.pallas_call(
        _tc1_body,
        grid=(_GRID,),
        in_specs=[pl.BlockSpec((_BLK, D), lambda i: (i, 0)),
                  pl.BlockSpec((D, D), lambda i: (0, 0)),
                  pl.BlockSpec((D, D), lambda i: (0, 0)),
                  pl.BlockSpec((1, D), lambda i: (0, 0))],
        out_specs=[pl.BlockSpec((_BLK, D), lambda i: (i, 0)),
                   pl.BlockSpec((_BLK, D), lambda i: (i, 0))],
        out_shape=[jax.ShapeDtypeStruct((N, D), jnp.float32),
                   jax.ShapeDtypeStruct((N, D), jnp.float32)],
    )(x, wl_t, wr_t, b)


def _tc2_body(sa_ref, sb_ref, ca_ref, cb_ref, r1_ref, wl_ref, wr_ref, b_ref,
              y_ref, r_ref):
    cnt = ca_ref[...] + cb_ref[...]
    mean = (sa_ref[...] + sb_ref[...]) / jnp.maximum(cnt, 1.0)
    h = jnp.maximum(mean + r1_ref[...], 0.0)
    y_ref[...] = jnp.dot(h, wl_ref[...], preferred_element_type=jnp.float32,
                         precision=lax.Precision.HIGHEST)
    r_ref[...] = jnp.dot(h, wr_ref[...], preferred_element_type=jnp.float32,
                         precision=lax.Precision.HIGHEST) + b_ref[...]


def _tc2(sa, sb, ca, cb, r1, wl_t, wr_t, b):
    return pl.pallas_call(
        _tc2_body,
        grid=(_GRID,),
        in_specs=[pl.BlockSpec((_BLK, D), lambda i: (i, 0)),
                  pl.BlockSpec((_BLK, D), lambda i: (i, 0)),
                  pl.BlockSpec((_BLK, 1), lambda i: (i, 0)),
                  pl.BlockSpec((_BLK, 1), lambda i: (i, 0)),
                  pl.BlockSpec((_BLK, D), lambda i: (i, 0)),
                  pl.BlockSpec((D, D), lambda i: (0, 0)),
                  pl.BlockSpec((D, D), lambda i: (0, 0)),
                  pl.BlockSpec((1, D), lambda i: (0, 0))],
        out_specs=[pl.BlockSpec((_BLK, D), lambda i: (i, 0)),
                   pl.BlockSpec((_BLK, D), lambda i: (i, 0))],
        out_shape=[jax.ShapeDtypeStruct((N, D), jnp.float32),
                   jax.ShapeDtypeStruct((N, D), jnp.float32)],
    )(sa, sb, ca, cb, r1, wl_t, wr_t, b)


def _tc3_body(sa_ref, sb_ref, ca_ref, cb_ref, r2_ref, o_ref):
    cnt = ca_ref[...] + cb_ref[...]
    mean = (sa_ref[...] + sb_ref[...]) / jnp.maximum(cnt, 1.0)
    z = mean + r2_ref[...]
    o_ref[...] = 1.0 / (1.0 + jnp.exp(-z))


def _tc3(sa, sb, ca, cb, r2):
    return pl.pallas_call(
        _tc3_body,
        grid=(_GRID,),
        in_specs=[pl.BlockSpec((_BLK, D), lambda i: (i, 0)),
                  pl.BlockSpec((_BLK, D), lambda i: (i, 0)),
                  pl.BlockSpec((_BLK, 1), lambda i: (i, 0)),
                  pl.BlockSpec((_BLK, 1), lambda i: (i, 0)),
                  pl.BlockSpec((_BLK, D), lambda i: (i, 0))],
        out_specs=pl.BlockSpec((_BLK, D), lambda i: (i, 0)),
        out_shape=jax.ShapeDtypeStruct((N, D), jnp.float32),
    )(sa, sb, ca, cb, r2)


def kernel(x, edge_index, W1_l, b1, W1_r, W2_l, b2, W2_r):
    pad = EP - E
    pad_src = jnp.arange(pad, dtype=jnp.int32) % N
    pad_dst = N + jnp.arange(pad, dtype=jnp.int32) % (NP - N)
    src_r = jnp.concatenate([edge_index[0], pad_src]).reshape(NW, NCH, CH)
    dst_r = jnp.concatenate([edge_index[1], pad_dst]).reshape(NW, NCH, CH)
    edges_r = jnp.stack([src_r, dst_r])
    zacc = jnp.zeros((NP, D), jnp.float32)
    y1, r1 = _tc1(x, W1_l.T, W1_r.T, b1.reshape(1, D))
    s1, cnt = _sc_agg(y1, edges_r, zacc, True)
    ca = cnt[:NP].reshape(NP, 1)
    cb = cnt[NP:].reshape(NP, 1)
    y2, r2 = _tc2(s1[0], s1[1], ca, cb, r1, W2_l.T, W2_r.T,
                  b2.reshape(1, D))
    s2, _cnt2 = _sc_agg(y2, edges_r, zacc, False)
    return _tc3(s2[0], s2[1], ca, cb, r2)


# speedup vs baseline: 8.1022x; 1.0160x over previous
import functools

import jax
import jax.numpy as jnp
from jax import lax
from jax.experimental import pallas as pl
from jax.experimental.pallas import tpu as pltpu
from jax.experimental.pallas import tpu_sc as plsc

N = 10000
E = 320000
D = 128
NC = 2
NS = 16
NW = NC * NS
EPW = 10240
CH = 128
NCH = EPW // CH
EP = NW * EPW
RPT = 632
NP = NS * RPT


def _sc_agg_body(with_cnt, y_hbm, edges_hbm, zacc_hbm,
                 sum_hbm, cnt_hbm,
                 src_v, dst_v, stage, ones_v, ctmp_v, acc_sh, cnt_sh,
                 gsem, ssem):
    c = lax.axis_index("c")
    s = lax.axis_index("s")
    wid = c * NS + s
    base = pl.multiple_of(s * RPT, 8)

    for k in range(CH // 16):
        ones_v[pl.ds(k * 16, 16)] = jnp.ones((16,), jnp.float32)
    for k in range(RPT // 16 + 1):
        ctmp_v[pl.ds(k * 16, 16)] = jnp.zeros((16,), jnp.float32)

    pltpu.sync_copy(zacc_hbm.at[pl.ds(base, RPT)], acc_sh.at[pl.ds(base, RPT)])
    if with_cnt:
        pltpu.sync_copy(ctmp_v.at[pl.ds(0, RPT)], cnt_sh.at[pl.ds(base, RPT)])

    pltpu.sync_copy(edges_hbm.at[0, wid], src_v)
    pltpu.sync_copy(edges_hbm.at[1, wid], dst_v)
    plsc.subcore_barrier()

    def chunk(p, carry):
        for u in range(2):
            j = 2 * p + u
            pltpu.async_copy(y_hbm.at[src_v.at[j]], stage, gsem).wait()
            pltpu.sync_copy(stage, acc_sh.at[dst_v.at[j]], add=True)
            if with_cnt:
                pltpu.sync_copy(ones_v, cnt_sh.at[dst_v.at[j]], add=True)
        return carry
    lax.fori_loop(0, NCH // 2, chunk, 0)

    plsc.subcore_barrier()

    pltpu.sync_copy(acc_sh.at[pl.ds(base, RPT)],
                    sum_hbm.at[c, pl.ds(base, RPT)])
    if with_cnt:
        pltpu.sync_copy(cnt_sh.at[pl.ds(base, RPT)], ctmp_v.at[pl.ds(0, RPT)])
        pltpu.sync_copy(ctmp_v.at[pl.ds(0, RPT)],
                        cnt_hbm.at[pl.ds(c * NP + base, RPT)])


def _sc_agg(y, edges_r, zacc, with_cnt):
    mesh = plsc.VectorSubcoreMesh(core_axis_name="c", subcore_axis_name="s")
    cnt_elems = NC * NP if with_cnt else 8
    out = pl.kernel(
        functools.partial(_sc_agg_body, with_cnt),
        out_type=(jax.ShapeDtypeStruct((NC, NP, D), jnp.float32),
                  jax.ShapeDtypeStruct((cnt_elems,), jnp.float32)),
        mesh=mesh,
        scratch_types=[
            pltpu.VMEM((NCH, CH), jnp.int32),
            pltpu.VMEM((NCH, CH), jnp.int32),
            pltpu.VMEM((CH, D), jnp.float32),
            pltpu.VMEM((CH,), jnp.float32),
            pltpu.VMEM((RPT + 16,), jnp.float32),
            pltpu.VMEM_SHARED((NP, D), jnp.float32),
            pltpu.VMEM_SHARED((NP,), jnp.float32),
            pltpu.SemaphoreType.DMA,
            pltpu.SemaphoreType.DMA,
        ],
    )(y, edges_r, zacc)
    return out


_BLK = 1000
_GRID = N // _BLK


def _tcy_body(x_ref, wl_ref, y_ref):
    y_ref[...] = jnp.dot(x_ref[...], wl_ref[...],
                         preferred_element_type=jnp.float32,
                         precision=lax.Precision.HIGHEST)


def _tcy(x, wl_t):
    return pl.pallas_call(
        _tcy_body,
        grid=(_GRID,),
        in_specs=[pl.BlockSpec((_BLK, D), lambda i: (i, 0)),
                  pl.BlockSpec((D, D), lambda i: (0, 0))],
        out_specs=pl.BlockSpec((_BLK, D), lambda i: (i, 0)),
        out_shape=jax.ShapeDtypeStruct((N, D), jnp.float32),
    )(x, wl_t)


def _tcr_body(x_ref, wr_ref, b_ref, r_ref):
    r_ref[...] = jnp.dot(x_ref[...], wr_ref[...],
                         preferred_element_type=jnp.float32,
                         precision=lax.Precision.HIGHEST) + b_ref[...]


def _tcr(x, wr_t, b):
    return pl.pallas_call(
        _tcr_body,
        grid=(_GRID,),
        in_specs=[pl.BlockSpec((_BLK, D), lambda i: (i, 0)),
                  pl.BlockSpec((D, D), lambda i: (0, 0)),
                  pl.BlockSpec((1, D), lambda i: (0, 0))],
        out_specs=pl.BlockSpec((_BLK, D), lambda i: (i, 0)),
        out_shape=jax.ShapeDtypeStruct((N, D), jnp.float32),
    )(x, wr_t, b)


def _tc2_body(sa_ref, sb_ref, ca_ref, cb_ref, r1_ref, wl_ref, wr_ref, b_ref,
              y_ref, r_ref):
    cnt = ca_ref[...] + cb_ref[...]
    mean = (sa_ref[...] + sb_ref[...]) / jnp.maximum(cnt, 1.0)
    h = jnp.maximum(mean + r1_ref[...], 0.0)
    y_ref[...] = jnp.dot(h, wl_ref[...], preferred_element_type=jnp.float32,
                         precision=lax.Precision.HIGHEST)
    r_ref[...] = jnp.dot(h, wr_ref[...], preferred_element_type=jnp.float32,
                         precision=lax.Precision.HIGHEST) + b_ref[...]


def _tc2(sa, sb, ca, cb, r1, wl_t, wr_t, b):
    return pl.pallas_call(
        _tc2_body,
        grid=(_GRID,),
        in_specs=[pl.BlockSpec((_BLK, D), lambda i: (i, 0)),
                  pl.BlockSpec((_BLK, D), lambda i: (i, 0)),
                  pl.BlockSpec((_BLK, 1), lambda i: (i, 0)),
                  pl.BlockSpec((_BLK, 1), lambda i: (i, 0)),
                  pl.BlockSpec((_BLK, D), lambda i: (i, 0)),
                  pl.BlockSpec((D, D), lambda i: (0, 0)),
                  pl.BlockSpec((D, D), lambda i: (0, 0)),
                  pl.BlockSpec((1, D), lambda i: (0, 0))],
        out_specs=[pl.BlockSpec((_BLK, D), lambda i: (i, 0)),
                   pl.BlockSpec((_BLK, D), lambda i: (i, 0))],
        out_shape=[jax.ShapeDtypeStruct((N, D), jnp.float32),
                   jax.ShapeDtypeStruct((N, D), jnp.float32)],
    )(sa, sb, ca, cb, r1, wl_t, wr_t, b)


def _tc3_body(sa_ref, sb_ref, ca_ref, cb_ref, r2_ref, o_ref):
    cnt = ca_ref[...] + cb_ref[...]
    mean = (sa_ref[...] + sb_ref[...]) / jnp.maximum(cnt, 1.0)
    z = mean + r2_ref[...]
    o_ref[...] = 1.0 / (1.0 + jnp.exp(-z))


def _tc3(sa, sb, ca, cb, r2):
    return pl.pallas_call(
        _tc3_body,
        grid=(_GRID,),
        in_specs=[pl.BlockSpec((_BLK, D), lambda i: (i, 0)),
                  pl.BlockSpec((_BLK, D), lambda i: (i, 0)),
                  pl.BlockSpec((_BLK, 1), lambda i: (i, 0)),
                  pl.BlockSpec((_BLK, 1), lambda i: (i, 0)),
                  pl.BlockSpec((_BLK, D), lambda i: (i, 0))],
        out_specs=pl.BlockSpec((_BLK, D), lambda i: (i, 0)),
        out_shape=jax.ShapeDtypeStruct((N, D), jnp.float32),
    )(sa, sb, ca, cb, r2)


def kernel(x, edge_index, W1_l, b1, W1_r, W2_l, b2, W2_r):
    pad = EP - E
    pad_src = jnp.arange(pad, dtype=jnp.int32) % N
    pad_dst = N + jnp.arange(pad, dtype=jnp.int32) % (NP - N)
    src_r = jnp.concatenate([edge_index[0], pad_src]).reshape(NW, NCH, CH)
    dst_r = jnp.concatenate([edge_index[1], pad_dst]).reshape(NW, NCH, CH)
    edges_r = jnp.stack([src_r, dst_r])
    zacc = jnp.zeros((NP, D), jnp.float32)
    y1 = _tcy(x, W1_l.T)
    s1, cnt = _sc_agg(y1, edges_r, zacc, True)
    r1 = _tcr(x, W1_r.T, b1.reshape(1, D))
    ca = cnt[:NP].reshape(NP, 1)
    cb = cnt[NP:].reshape(NP, 1)
    y2, r2 = _tc2(s1[0], s1[1], ca, cb, r1, W2_l.T, W2_r.T,
                  b2.reshape(1, D))
    s2, _cnt2 = _sc_agg(y2, edges_r, zacc, False)
    return _tc3(s2[0], s2[1], ca, cb, r2)


# speedup vs baseline: 8.2477x; 1.0180x over previous
import functools

import jax
import jax.numpy as jnp
from jax import lax
from jax.experimental import pallas as pl
from jax.experimental.pallas import tpu as pltpu
from jax.experimental.pallas import tpu_sc as plsc

N = 10000
E = 320000
D = 128
NC = 2
NS = 16
NW = NC * NS
EPW = 10240
CH = 128
NCH = EPW // CH
EP = NW * EPW
RPT = 632
NP = NS * RPT


def _sc_agg_body(with_cnt, y_hbm, edges_hbm, zacc_hbm,
                 sum_hbm, cnt_hbm,
                 src_v, dst_v, stage, ones_v, ctmp_v, acc_sh, cnt_sh,
                 gsem, ssem):
    c = lax.axis_index("c")
    s = lax.axis_index("s")
    wid = c * NS + s
    base = pl.multiple_of(s * RPT, 8)

    for k in range(CH // 16):
        ones_v[pl.ds(k * 16, 16)] = jnp.ones((16,), jnp.float32)
    for k in range(RPT // 16 + 1):
        ctmp_v[pl.ds(k * 16, 16)] = jnp.zeros((16,), jnp.float32)

    pltpu.sync_copy(zacc_hbm.at[pl.ds(base, RPT)], acc_sh.at[pl.ds(base, RPT)])
    if with_cnt:
        pltpu.sync_copy(ctmp_v.at[pl.ds(0, RPT)], cnt_sh.at[pl.ds(base, RPT)])

    pltpu.sync_copy(edges_hbm.at[0, wid], src_v)
    pltpu.sync_copy(edges_hbm.at[1, wid], dst_v)
    plsc.subcore_barrier()

    def chunk(p, carry):
        for u in range(2):
            j = 2 * p + u
            pltpu.async_copy(y_hbm.at[src_v.at[j]], stage, gsem).wait()
            pltpu.sync_copy(stage, acc_sh.at[dst_v.at[j]], add=True)
            if with_cnt:
                pltpu.sync_copy(ones_v, cnt_sh.at[dst_v.at[j]], add=True)
        return carry
    lax.fori_loop(0, NCH // 2, chunk, 0)

    plsc.subcore_barrier()

    pltpu.sync_copy(acc_sh.at[pl.ds(base, RPT)],
                    sum_hbm.at[c, pl.ds(base, RPT)])
    if with_cnt:
        pltpu.sync_copy(cnt_sh.at[pl.ds(base, RPT)], ctmp_v.at[pl.ds(0, RPT)])
        pltpu.sync_copy(ctmp_v.at[pl.ds(0, RPT)],
                        cnt_hbm.at[pl.ds(c * NP + base, RPT)])


def _sc_agg(y, edges_r, zacc, with_cnt):
    mesh = plsc.VectorSubcoreMesh(core_axis_name="c", subcore_axis_name="s")
    cnt_elems = NC * NP if with_cnt else 8
    out = pl.kernel(
        functools.partial(_sc_agg_body, with_cnt),
        out_type=(jax.ShapeDtypeStruct((NC, NP, D), jnp.float32),
                  jax.ShapeDtypeStruct((cnt_elems,), jnp.float32)),
        mesh=mesh,
        scratch_types=[
            pltpu.VMEM((NCH, CH), jnp.int32),
            pltpu.VMEM((NCH, CH), jnp.int32),
            pltpu.VMEM((CH, D), jnp.float32),
            pltpu.VMEM((CH,), jnp.float32),
            pltpu.VMEM((RPT + 16,), jnp.float32),
            pltpu.VMEM_SHARED((NP, D), jnp.float32),
            pltpu.VMEM_SHARED((NP,), jnp.float32),
            pltpu.SemaphoreType.DMA,
            pltpu.SemaphoreType.DMA,
        ],
    )(y, edges_r, zacc)
    return out


_BLK = 1000
_GRID = N // _BLK


def _tcy_body(x_ref, wl_ref, y_ref):
    y_ref[...] = jnp.dot(x_ref[...], wl_ref[...],
                         preferred_element_type=jnp.float32,
                         precision=lax.Precision.HIGHEST)


def _tcy(x, wl_t):
    return pl.pallas_call(
        _tcy_body,
        grid=(_GRID,),
        in_specs=[pl.BlockSpec((_BLK, D), lambda i: (i, 0)),
                  pl.BlockSpec((D, D), lambda i: (0, 0))],
        out_specs=pl.BlockSpec((_BLK, D), lambda i: (i, 0)),
        out_shape=jax.ShapeDtypeStruct((N, D), jnp.float32),
    )(x, wl_t)


def _tcr_body(x_ref, wr_ref, b_ref, r_ref):
    r_ref[...] = jnp.dot(x_ref[...], wr_ref[...],
                         preferred_element_type=jnp.float32,
                         precision=lax.Precision.HIGHEST) + b_ref[...]


def _tcr(x, wr_t, b):
    return pl.pallas_call(
        _tcr_body,
        grid=(_GRID,),
        in_specs=[pl.BlockSpec((_BLK, D), lambda i: (i, 0)),
                  pl.BlockSpec((D, D), lambda i: (0, 0)),
                  pl.BlockSpec((1, D), lambda i: (0, 0))],
        out_specs=pl.BlockSpec((_BLK, D), lambda i: (i, 0)),
        out_shape=jax.ShapeDtypeStruct((N, D), jnp.float32),
    )(x, wr_t, b)


def _tc2_body(sa_ref, sb_ref, ca_ref, cb_ref, x_ref, wl_ref, wr_ref, b_ref,
              h_ref):
    cnt = ca_ref[...] + cb_ref[...]
    mean = (sa_ref[...] + sb_ref[...]) / jnp.maximum(cnt, 1.0)
    z = (jnp.dot(mean, wl_ref[...], preferred_element_type=jnp.float32,
                 precision=lax.Precision.HIGHEST) + b_ref[...]
         + jnp.dot(x_ref[...], wr_ref[...], preferred_element_type=jnp.float32,
                   precision=lax.Precision.HIGHEST))
    h_ref[...] = jnp.maximum(z, 0.0)


def _tc2(sa, sb, ca, cb, x, wl_t, wr_t, b):
    return pl.pallas_call(
        _tc2_body,
        grid=(_GRID,),
        in_specs=[pl.BlockSpec((_BLK, D), lambda i: (i, 0)),
                  pl.BlockSpec((_BLK, D), lambda i: (i, 0)),
                  pl.BlockSpec((_BLK, 1), lambda i: (i, 0)),
                  pl.BlockSpec((_BLK, 1), lambda i: (i, 0)),
                  pl.BlockSpec((_BLK, D), lambda i: (i, 0)),
                  pl.BlockSpec((D, D), lambda i: (0, 0)),
                  pl.BlockSpec((D, D), lambda i: (0, 0)),
                  pl.BlockSpec((1, D), lambda i: (0, 0))],
        out_specs=pl.BlockSpec((_BLK, D), lambda i: (i, 0)),
        out_shape=jax.ShapeDtypeStruct((N, D), jnp.float32),
    )(sa, sb, ca, cb, x, wl_t, wr_t, b)


def _tc3_body(sa_ref, sb_ref, ca_ref, cb_ref, h_ref, wl_ref, wr_ref, b_ref,
              o_ref):
    cnt = ca_ref[...] + cb_ref[...]
    mean = (sa_ref[...] + sb_ref[...]) / jnp.maximum(cnt, 1.0)
    z = (jnp.dot(mean, wl_ref[...], preferred_element_type=jnp.float32,
                 precision=lax.Precision.HIGHEST) + b_ref[...]
         + jnp.dot(h_ref[...], wr_ref[...], preferred_element_type=jnp.float32,
                   precision=lax.Precision.HIGHEST))
    o_ref[...] = 1.0 / (1.0 + jnp.exp(-z))


def _tc3(sa, sb, ca, cb, h, wl_t, wr_t, b):
    return pl.pallas_call(
        _tc3_body,
        grid=(_GRID,),
        in_specs=[pl.BlockSpec((_BLK, D), lambda i: (i, 0)),
                  pl.BlockSpec((_BLK, D), lambda i: (i, 0)),
                  pl.BlockSpec((_BLK, 1), lambda i: (i, 0)),
                  pl.BlockSpec((_BLK, 1), lambda i: (i, 0)),
                  pl.BlockSpec((_BLK, D), lambda i: (i, 0)),
                  pl.BlockSpec((D, D), lambda i: (0, 0)),
                  pl.BlockSpec((D, D), lambda i: (0, 0)),
                  pl.BlockSpec((1, D), lambda i: (0, 0))],
        out_specs=pl.BlockSpec((_BLK, D), lambda i: (i, 0)),
        out_shape=jax.ShapeDtypeStruct((N, D), jnp.float32),
    )(sa, sb, ca, cb, h, wl_t, wr_t, b)


def kernel(x, edge_index, W1_l, b1, W1_r, W2_l, b2, W2_r):
    pad = EP - E
    pad_src = jnp.arange(pad, dtype=jnp.int32) % N
    pad_dst = N + jnp.arange(pad, dtype=jnp.int32) % (NP - N)
    src_r = jnp.concatenate([edge_index[0], pad_src]).reshape(NW, NCH, CH)
    dst_r = jnp.concatenate([edge_index[1], pad_dst]).reshape(NW, NCH, CH)
    edges_r = jnp.stack([src_r, dst_r])
    zacc = jnp.zeros((NP, D), jnp.float32)
    s1, cnt = _sc_agg(x, edges_r, zacc, True)
    ca = cnt[:NP].reshape(NP, 1)
    cb = cnt[NP:].reshape(NP, 1)
    h = _tc2(s1[0], s1[1], ca, cb, x, W1_l.T, W1_r.T, b1.reshape(1, D))
    s2, _cnt2 = _sc_agg(h, edges_r, zacc, False)
    return _tc3(s2[0], s2[1], ca, cb, h, W2_l.T, W2_r.T, b2.reshape(1, D))
